# baseline (speedup 1.0000x reference)
import functools

import jax
import jax.numpy as jnp
from jax import lax
from jax.experimental import pallas as pl
from jax.experimental.pallas import tpu as pltpu
from jax.experimental.pallas import tpu_sc as plsc

SEQ = 8192
DIM = 4096
NUM_CORES = 2
NUM_SUBCORES = 16
NUM_WORKERS = NUM_CORES * NUM_SUBCORES
ROWS_PER_WORKER = SEQ // NUM_WORKERS

ROWS_PER_CHUNK = 2
NBUF = 8
LAG = NBUF // 2
NCHUNK = ROWS_PER_WORKER // ROWS_PER_CHUNK
HALF = NCHUNK // 2

_mesh = plsc.VectorSubcoreMesh(
    core_axis_name="c", subcore_axis_name="s", num_cores=NUM_CORES
)


@functools.partial(
    pl.kernel,
    out_type=jax.ShapeDtypeStruct((SEQ, DIM), jnp.float32),
    mesh=_mesh,
    scratch_types=[
        pltpu.VMEM((NBUF, ROWS_PER_CHUNK, DIM), jnp.float32),
        pltpu.VMEM_SHARED((NUM_SUBCORES, NBUF, ROWS_PER_CHUNK, DIM), jnp.float32),
        pltpu.SemaphoreType.DMA((NBUF,)),
        pltpu.SemaphoreType.DMA((NBUF,)),
        pltpu.SemaphoreType.DMA((NBUF,)),
        pltpu.SemaphoreType.DMA((NBUF,)),
    ],
)
def _sc_identity_gather(
    table_hbm, out_hbm, tile_bufs, shared_bufs, vl_sems, vs_sems, sl_sems, ss_sems
):
    wid = lax.axis_index("s") * NUM_CORES + lax.axis_index("c")
    wbase = wid * ROWS_PER_WORKER
    spmem_bufs = shared_bufs.at[lax.axis_index("s")]

    def make_path(bufs, load_sems, store_sems, chunk0):
        def load_desc(i, b):
            row = wbase + (chunk0 + i) * ROWS_PER_CHUNK
            return pltpu.make_async_copy(
                table_hbm.at[pl.ds(row, ROWS_PER_CHUNK)], bufs.at[b], load_sems.at[b]
            )

        def store_desc(i, b):
            row = wbase + (chunk0 + i) * ROWS_PER_CHUNK
            return pltpu.make_async_copy(
                bufs.at[b], out_hbm.at[pl.ds(row, ROWS_PER_CHUNK)], store_sems.at[b]
            )

        return load_desc, store_desc

    paths = [
        make_path(tile_bufs, vl_sems, vs_sems, 0),
        make_path(spmem_bufs, sl_sems, ss_sems, HALF),
    ]

    for load_desc, _ in paths:
        for b in range(NBUF):
            load_desc(b, b).start()

    for i in range(LAG):
        for load_desc, store_desc in paths:
            load_desc(i, i % NBUF).wait()
            store_desc(i, i % NBUF).start()

    def step(i, carry):
        for load_desc, store_desc in paths:
            b = i % NBUF
            load_desc(i, b).wait()
            store_desc(i, b).start()
            j = i - LAG
            bj = j % NBUF
            store_desc(j, bj).wait()
            load_desc(j + NBUF, bj).start()
        return carry

    lax.fori_loop(LAG, HALF - LAG, step, 0)

    for i in range(HALF - LAG, HALF):
        for load_desc, store_desc in paths:
            b = i % NBUF
            load_desc(i, b).wait()
            store_desc(i, b).start()
            store_desc(i - LAG, (i - LAG) % NBUF).wait()

    for i in range(HALF - LAG, HALF):
        for _, store_desc in paths:
            store_desc(i, i % NBUF).wait()


def kernel(x, emb_weight):
    del x
    return _sc_identity_gather(emb_weight)


# speedup vs baseline: 1.0145x; 1.0145x over previous
import functools

import jax
import jax.numpy as jnp
from jax import lax
from jax.experimental import pallas as pl
from jax.experimental.pallas import tpu as pltpu
from jax.experimental.pallas import tpu_sc as plsc

SEQ = 8192
DIM = 4096
NUM_CORES = 2
NUM_SUBCORES = 16
NUM_WORKERS = NUM_CORES * NUM_SUBCORES
ROWS_PER_WORKER = SEQ // NUM_WORKERS

ROWS_PER_CHUNK = 4
NBUF = 6
LAG = NBUF // 2
NCHUNK = ROWS_PER_WORKER // ROWS_PER_CHUNK

_mesh = plsc.VectorSubcoreMesh(
    core_axis_name="c", subcore_axis_name="s", num_cores=NUM_CORES
)


@functools.partial(
    pl.kernel,
    out_type=jax.ShapeDtypeStruct((SEQ, DIM), jnp.float32),
    mesh=_mesh,
    scratch_types=[
        pltpu.VMEM_SHARED((NUM_SUBCORES, NBUF, ROWS_PER_CHUNK, DIM), jnp.float32),
        pltpu.SemaphoreType.DMA((NBUF,)),
        pltpu.SemaphoreType.DMA((NBUF,)),
    ],
)
def _sc_identity_gather(table_hbm, out_hbm, shared_bufs, load_sems, store_sems):
    wid = lax.axis_index("s") * NUM_CORES + lax.axis_index("c")
    wbase = wid * ROWS_PER_WORKER
    bufs = shared_bufs.at[lax.axis_index("s")]

    def load_desc(i, b):
        row = wbase + i * ROWS_PER_CHUNK
        return pltpu.make_async_copy(
            table_hbm.at[pl.ds(row, ROWS_PER_CHUNK)], bufs.at[b], load_sems.at[b]
        )

    def store_desc(i, b):
        row = wbase + i * ROWS_PER_CHUNK
        return pltpu.make_async_copy(
            bufs.at[b], out_hbm.at[pl.ds(row, ROWS_PER_CHUNK)], store_sems.at[b]
        )

    for b in range(NBUF):
        load_desc(b, b).start()

    for i in range(LAG):
        load_desc(i, i % NBUF).wait()
        store_desc(i, i % NBUF).start()

    def step(i, carry):
        b = i % NBUF
        load_desc(i, b).wait()
        store_desc(i, b).start()
        j = i - LAG
        bj = j % NBUF
        store_desc(j, bj).wait()
        load_desc(j + NBUF, bj).start()
        return carry

    lax.fori_loop(LAG, NCHUNK - LAG, step, 0)

    for i in range(NCHUNK - LAG, NCHUNK):
        b = i % NBUF
        load_desc(i, b).wait()
        store_desc(i, b).start()
        store_desc(i - LAG, (i - LAG) % NBUF).wait()

    for i in range(NCHUNK - LAG, NCHUNK):
        store_desc(i, i % NBUF).wait()


def kernel(x, emb_weight):
    del x
    return _sc_identity_gather(emb_weight)
